# Initial kernel scaffold; baseline (speedup 1.0000x reference)
import jax
import jax.numpy as jnp
from jax.experimental import pallas as pl


def kernel(x, edge_index, W, att_src, att_dst, bias, bn_gamma, bn_beta):
    raise NotImplementedError("write your pallas kernel here")


# speedup vs baseline: 10.2746x; 10.2746x over previous
import jax
import jax.numpy as jnp
from jax import lax
from jax.experimental import pallas as pl
from jax.experimental.pallas import tpu as pltpu
from jax.experimental.pallas import tpu_sc as plsc

NEG = 0.2
_INTERP = False
B = 16
NTILES = 16
CW = 64
NCH = 4


def _mm_body(x_ref, w_ref, o_ref):
    y = jnp.dot(x_ref[...], w_ref[...], preferred_element_type=jnp.float32)
    o_ref[0] = y[:, :CW]
    o_ref[1] = y[:, CW:]


def _bcast_lane(v, i):
    dnums = lax.GatherDimensionNumbers(
        offset_dims=(), collapsed_slice_dims=(0,), start_index_map=(0,))
    idx = jnp.full((16, 1), i, jnp.int32)
    return lax.gather(v, idx, dnums, (1,),
                      mode=lax.GatherScatterMode.PROMISE_IN_BOUNDS)


def _make_sc_kernel(NT, NBLK):
    RPT = NT // NTILES
    EPT = NBLK * B

    def _sc_body(h4, asd, srcr, dstr, o4, dn,
                 src_v, dst_v, ex_v, as_t, ad_t, rows, zbuf, zrow,
                 out_acc, den_acc, sem_g, sem_s, sem_d):
        c = lax.axis_index("c")
        s = lax.axis_index("s")

        pltpu.sync_copy(srcr.at[pl.ds(s * EPT, EPT)], src_v)
        pltpu.sync_copy(dstr.at[pl.ds(s * EPT, EPT)], dst_v)
        pltpu.sync_copy(asd.at[pl.ds(c * NT, NT)], as_t)
        pltpu.sync_copy(asd.at[pl.ds((c + 2) * NT, NT)], ad_t)

        zero16 = jnp.zeros((16,), jnp.float32)
        for i in range(64):
            for q in range(CW // 16):
                zbuf[i, pl.ds(q * 16, 16)] = zero16
        for i in range(RPT // 16):
            zrow[pl.ds(i * 16, 16)] = zero16
        pltpu.sync_copy(zrow, den_acc.at[pl.ds(s * RPT, RPT)])

        for jj in range(NCH):
            plane = c * NCH + jj
            for k in range(RPT // 64):
                pltpu.sync_copy(zbuf, out_acc.at[pl.ds(s * RPT + k * 64, 64)])
            plsc.subcore_barrier()

            def block_body(b, carry):
                si = src_v[pl.ds(b * B, B)]
                di = dst_v[pl.ds(b * B, B)]
                pltpu.async_copy(h4.at[plane].at[si], rows, sem_g).wait()
                if jj == 0:
                    asv = plsc.load_gather(as_t, [si])
                    adv = plsc.load_gather(ad_t, [di])
                    e = asv + adv
                    e = jnp.where(e > 0, e, NEG * e)
                    ex = jnp.exp(e)
                    ex_v[pl.ds(b * B, B)] = ex
                    pltpu.async_copy(ex_v.at[pl.ds(b * B, B)],
                                     den_acc.at[di], sem_d, add=True).wait()
                else:
                    ex = ex_v[pl.ds(b * B, B)]
                for i in range(B):
                    bi = _bcast_lane(ex, i)
                    for q in range(CW // 16):
                        sl = pl.ds(q * 16, 16)
                        rows[i, sl] = rows[i, sl] * bi
                pltpu.async_copy(rows, out_acc.at[di], sem_s, add=True).wait()
                return carry

            lax.fori_loop(0, NBLK, block_body, 0)
            plsc.subcore_barrier()
            pltpu.sync_copy(out_acc.at[pl.ds(s * RPT, RPT)],
                            o4.at[plane].at[pl.ds(s * RPT, RPT)])
            if jj == 0:
                pltpu.sync_copy(den_acc.at[pl.ds(s * RPT, RPT)],
                                dn.at[c].at[pl.ds(s * RPT, RPT)])
            plsc.subcore_barrier()

    mesh = plsc.VectorSubcoreMesh(core_axis_name="c", subcore_axis_name="s",
                                  num_cores=2, num_subcores=NTILES)
    return pl.kernel(
        _sc_body,
        out_type=[jax.ShapeDtypeStruct((2 * NCH, NT, CW), jnp.float32),
                  jax.ShapeDtypeStruct((2, NT), jnp.float32)],
        mesh=mesh,
        interpret=_INTERP,
        compiler_params=pltpu.CompilerParams(
            needs_layout_passes=False, use_tc_tiling_on_sc=False),
        scratch_types=[
            pltpu.VMEM((EPT,), jnp.int32),
            pltpu.VMEM((EPT,), jnp.int32),
            pltpu.VMEM((EPT,), jnp.float32),
            pltpu.VMEM((NT,), jnp.float32),
            pltpu.VMEM((NT,), jnp.float32),
            pltpu.VMEM((B, CW), jnp.float32),
            pltpu.VMEM((64, CW), jnp.float32),
            pltpu.VMEM((NT // NTILES,), jnp.float32),
            pltpu.VMEM_SHARED((NT, CW), jnp.float32),
            pltpu.VMEM_SHARED((NT,), jnp.float32),
            pltpu.SemaphoreType.DMA,
            pltpu.SemaphoreType.DMA,
            pltpu.SemaphoreType.DMA,
        ],
    )


def _make_bn_body(NT, n):
    def _bn_body(o4_ref, den_ref, bgb_ref, out_ref, st_ref):
        p = pl.program_id(1)
        den = den_ref[0]
        bias = bgb_ref[0, 0, 0:1, :]
        deni = 1.0 / (den + 1e-16)
        tmp = jnp.concatenate([o4_ref[0] * deni, o4_ref[1] * deni],
                              axis=1) + bias
        rowid = lax.broadcasted_iota(jnp.int32, (NT, 1), 0)
        tmpm = jnp.where(rowid < n, tmp, 0.0)

        @pl.when(p == 0)
        def _():
            st_ref[0:1, :] = jnp.sum(tmpm, axis=0, keepdims=True)
            st_ref[1:2, :] = jnp.sum(tmpm * tmpm, axis=0, keepdims=True)

        @pl.when(p == 1)
        def _():
            mean = st_ref[0:1, :] / n
            var = st_ref[1:2, :] / n - mean * mean
            gamma = bgb_ref[1, 0, 0:1, :]
            beta = bgb_ref[2, 0, 0:1, :]
            y = (tmp - mean) * lax.rsqrt(var + 1e-5) * gamma + beta
            out_ref[...] = jnp.maximum(y[:n, :], 0.0)

    return _bn_body


@jax.jit
def kernel(x, edge_index, W, att_src, att_dst, bias, bn_gamma, bn_beta):
    n, IN = x.shape
    H = att_src.shape[0]
    OUT = att_src.shape[1]
    e = edge_index.shape[1]
    NT = ((n + 255) // 256) * 256
    EE = e + n
    NBLK = -(-EE // (NTILES * B))
    NBLK = ((NBLK + 3) // 4) * 4
    EP = NTILES * NBLK * B

    W3 = W.reshape(IN, H, OUT)
    Ws = (W3 * att_src[None]).sum(-1)
    Wd = (W3 * att_dst[None]).sum(-1)
    extra = jnp.zeros((IN, 2 * CW), jnp.float32)
    extra = extra.at[:, 0:H].set(Ws).at[:, H:2 * H].set(Wd)
    w_aug = jnp.concatenate([W, extra], axis=1)
    x_pad = jnp.pad(x, ((0, NT - n), (0, 0)))

    NCB = w_aug.shape[1] // CW
    y = pl.pallas_call(
        _mm_body,
        grid=(NT // 1024, NCB // 2),
        in_specs=[pl.BlockSpec((1024, IN), lambda i, j: (i, 0)),
                  pl.BlockSpec((IN, 2 * CW), lambda i, j: (0, j))],
        out_specs=pl.BlockSpec((2, 1024, CW), lambda i, j: (j, i, 0)),
        out_shape=jax.ShapeDtypeStruct((NCB, NT, CW), jnp.float32),
    )(x_pad, w_aug)

    asd = y[2 * NCH, :, 0:2 * H].T.reshape(-1)

    loops = jnp.arange(n, dtype=jnp.int32)
    src = jnp.concatenate([edge_index[0], loops])
    dst = jnp.concatenate([edge_index[1], loops])
    padrow = n + (jnp.arange(EP - EE, dtype=jnp.int32) % (NT - n))
    srcr = jnp.concatenate([src, padrow])
    dstr = jnp.concatenate([dst, padrow])

    o4, dn = _make_sc_kernel(NT, NBLK)(y, asd, srcr, dstr)

    denb = jnp.broadcast_to(dn[:, :, None], (H, NT, CW))
    bgb = jnp.stack([bias, bn_gamma, bn_beta]).reshape(3, NCH, 1, 2 * CW)
    bgb = jnp.broadcast_to(bgb, (3, NCH, 8, 2 * CW))

    out = pl.pallas_call(
        _make_bn_body(NT, n),
        grid=(NCH, 2),
        in_specs=[pl.BlockSpec((2, NT, CW), lambda c4, p: (c4, 0, 0)),
                  pl.BlockSpec((1, NT, CW), lambda c4, p: (c4 // 2, 0, 0)),
                  pl.BlockSpec((3, 1, 8, 2 * CW),
                               lambda c4, p: (0, c4, 0, 0))],
        out_specs=pl.BlockSpec((n, 2 * CW), lambda c4, p: (0, c4)),
        out_shape=jax.ShapeDtypeStruct((n, H * OUT), jnp.float32),
        scratch_shapes=[pltpu.VMEM((8, 2 * CW), jnp.float32)],
    )(o4, denb, bgb)
    return out


# speedup vs baseline: 31.3207x; 3.0484x over previous
import jax
import jax.numpy as jnp
from jax import lax
from jax.experimental import pallas as pl
from jax.experimental.pallas import tpu as pltpu
from jax.experimental.pallas import tpu_sc as plsc

NEG = 0.2
_INTERP = False
B = 16
NB = 4
NTILES = 16
CW = 64
NCH = 4


def _mm_body(x_ref, w_ref, o_ref):
    y = jnp.dot(x_ref[...], w_ref[...], preferred_element_type=jnp.float32)
    o_ref[0] = y[:, :CW]
    o_ref[1] = y[:, CW:]


def _bcast_lane(v, i):
    dnums = lax.GatherDimensionNumbers(
        offset_dims=(), collapsed_slice_dims=(0,), start_index_map=(0,))
    idx = jnp.full((16, 1), i, jnp.int32)
    return lax.gather(v, idx, dnums, (1,),
                      mode=lax.GatherScatterMode.PROMISE_IN_BOUNDS)


def _make_sc_kernel(NT, NBLK):
    RPT = NT // NTILES
    EPT = NBLK * B

    def _sc_body(h4, asd, srcr, dstr, o4, dn,
                 src_v, dst_v, ex_v, as_t, ad_t, rows, sbuf, zbuf, zrow,
                 out_acc, den_acc, sem_g, sem_s, sem_d):
        c = lax.axis_index("c")
        s = lax.axis_index("s")

        pltpu.sync_copy(srcr.at[pl.ds(s * EPT, EPT)], src_v)
        pltpu.sync_copy(dstr.at[pl.ds(s * EPT, EPT)], dst_v)
        pltpu.sync_copy(asd.at[pl.ds(c * NT, NT)], as_t)
        pltpu.sync_copy(asd.at[pl.ds((c + 2) * NT, NT)], ad_t)

        zero16 = jnp.zeros((16,), jnp.float32)
        for i in range(64):
            for q in range(CW // 16):
                zbuf[i, pl.ds(q * 16, 16)] = zero16
        for i in range(RPT // 16):
            zrow[pl.ds(i * 16, 16)] = zero16
        pltpu.sync_copy(zrow, den_acc.at[pl.ds(s * RPT, RPT)])

        NGRP = NBLK // NB
        for jj in range(NCH):
            plane = c * NCH + jj
            for k in range(RPT // 64):
                pltpu.sync_copy(zbuf, out_acc.at[pl.ds(s * RPT + k * 64, 64)])
            plsc.subcore_barrier()

            for k in range(NB):
                pltpu.async_copy(h4.at[plane].at[src_v[pl.ds(k * B, B)]],
                                 rows.at[k], sem_g.at[k])

            def group_body(g, carry):
                for k in range(NB):
                    b = g * NB + k
                    di = dst_v[pl.ds(b * B, B)]
                    pltpu.make_async_copy(h4.at[plane].at[di], rows.at[k],
                                          sem_g.at[k]).wait()
                    if jj == 0:
                        si = src_v[pl.ds(b * B, B)]
                        asv = plsc.load_gather(as_t, [si])
                        adv = plsc.load_gather(ad_t, [di])
                        e = asv + adv
                        e = jnp.where(e > 0, e, NEG * e)
                        ex = jnp.exp(e)
                        ex_v[pl.ds(b * B, B)] = ex
                        pltpu.async_copy(ex_v.at[pl.ds(b * B, B)],
                                         den_acc.at[di], sem_d,
                                         add=True).wait()
                    else:
                        ex = ex_v[pl.ds(b * B, B)]

                    @pl.when(g > 0)
                    def _():
                        pltpu.make_async_copy(sbuf.at[k], out_acc.at[di],
                                              sem_s.at[k]).wait()

                    for i in range(B):
                        bi = _bcast_lane(ex, i)
                        for q in range(CW // 16):
                            sl = pl.ds(q * 16, 16)
                            sbuf[k, i, sl] = rows[k, i, sl] * bi

                    @pl.when(g < NGRP - 1)
                    def _():
                        si2 = src_v[pl.ds((b + NB) * B, B)]
                        pltpu.async_copy(h4.at[plane].at[si2], rows.at[k],
                                         sem_g.at[k])

                    pltpu.async_copy(sbuf.at[k], out_acc.at[di], sem_s.at[k],
                                     add=True)
                return carry

            lax.fori_loop(0, NGRP, group_body, 0)
            for k in range(NB):
                pltpu.make_async_copy(sbuf.at[k],
                                      out_acc.at[dst_v[pl.ds(k * B, B)]],
                                      sem_s.at[k]).wait()
            plsc.subcore_barrier()
            pltpu.sync_copy(out_acc.at[pl.ds(s * RPT, RPT)],
                            o4.at[plane].at[pl.ds(s * RPT, RPT)])
            if jj == 0:
                pltpu.sync_copy(den_acc.at[pl.ds(s * RPT, RPT)],
                                dn.at[c].at[pl.ds(s * RPT, RPT)])
            plsc.subcore_barrier()

    mesh = plsc.VectorSubcoreMesh(core_axis_name="c", subcore_axis_name="s",
                                  num_cores=2, num_subcores=NTILES)
    return pl.kernel(
        _sc_body,
        out_type=[jax.ShapeDtypeStruct((2 * NCH, NT, CW), jnp.float32),
                  jax.ShapeDtypeStruct((2, NT), jnp.float32)],
        mesh=mesh,
        interpret=_INTERP,
        compiler_params=pltpu.CompilerParams(
            needs_layout_passes=False, use_tc_tiling_on_sc=False),
        scratch_types=[
            pltpu.VMEM((EPT,), jnp.int32),
            pltpu.VMEM((EPT,), jnp.int32),
            pltpu.VMEM((EPT,), jnp.float32),
            pltpu.VMEM((NT,), jnp.float32),
            pltpu.VMEM((NT,), jnp.float32),
            pltpu.VMEM((NB, B, CW), jnp.float32),
            pltpu.VMEM((NB, B, CW), jnp.float32),
            pltpu.VMEM((64, CW), jnp.float32),
            pltpu.VMEM((NT // NTILES,), jnp.float32),
            pltpu.VMEM_SHARED((NT, CW), jnp.float32),
            pltpu.VMEM_SHARED((NT,), jnp.float32),
            pltpu.SemaphoreType.DMA((NB,)),
            pltpu.SemaphoreType.DMA((NB,)),
            pltpu.SemaphoreType.DMA,
        ],
    )


def _make_bn_body(NT, n):
    def _bn_body(o4_ref, den_ref, bgb_ref, out_ref, st_ref):
        p = pl.program_id(1)
        den = den_ref[0]
        bias = bgb_ref[0, 0, 0:1, :]
        deni = 1.0 / (den + 1e-16)
        tmp = jnp.concatenate([o4_ref[0] * deni, o4_ref[1] * deni],
                              axis=1) + bias
        rowid = lax.broadcasted_iota(jnp.int32, (NT, 1), 0)
        tmpm = jnp.where(rowid < n, tmp, 0.0)

        @pl.when(p == 0)
        def _():
            st_ref[0:1, :] = jnp.sum(tmpm, axis=0, keepdims=True)
            st_ref[1:2, :] = jnp.sum(tmpm * tmpm, axis=0, keepdims=True)

        @pl.when(p == 1)
        def _():
            mean = st_ref[0:1, :] / n
            var = st_ref[1:2, :] / n - mean * mean
            gamma = bgb_ref[1, 0, 0:1, :]
            beta = bgb_ref[2, 0, 0:1, :]
            y = (tmp - mean) * lax.rsqrt(var + 1e-5) * gamma + beta
            out_ref[...] = jnp.maximum(y[:n, :], 0.0)

    return _bn_body


@jax.jit
def kernel(x, edge_index, W, att_src, att_dst, bias, bn_gamma, bn_beta):
    n, IN = x.shape
    H = att_src.shape[0]
    OUT = att_src.shape[1]
    e = edge_index.shape[1]
    NT = ((n + 255) // 256) * 256
    EE = e + n
    NBLK = -(-EE // (NTILES * B))
    NBLK = ((NBLK + 3) // 4) * 4
    EP = NTILES * NBLK * B

    W3 = W.reshape(IN, H, OUT)
    Ws = (W3 * att_src[None]).sum(-1)
    Wd = (W3 * att_dst[None]).sum(-1)
    extra = jnp.zeros((IN, 2 * CW), jnp.float32)
    extra = extra.at[:, 0:H].set(Ws).at[:, H:2 * H].set(Wd)
    w_aug = jnp.concatenate([W, extra], axis=1)
    x_pad = jnp.pad(x, ((0, NT - n), (0, 0)))

    NCB = w_aug.shape[1] // CW
    y = pl.pallas_call(
        _mm_body,
        grid=(NT // 1024, NCB // 2),
        in_specs=[pl.BlockSpec((1024, IN), lambda i, j: (i, 0)),
                  pl.BlockSpec((IN, 2 * CW), lambda i, j: (0, j))],
        out_specs=pl.BlockSpec((2, 1024, CW), lambda i, j: (j, i, 0)),
        out_shape=jax.ShapeDtypeStruct((NCB, NT, CW), jnp.float32),
    )(x_pad, w_aug)

    asd = y[2 * NCH, :, 0:2 * H].T.reshape(-1)

    loops = jnp.arange(n, dtype=jnp.int32)
    src = jnp.concatenate([edge_index[0], loops])
    dst = jnp.concatenate([edge_index[1], loops])
    padrow = n + (jnp.arange(EP - EE, dtype=jnp.int32) % (NT - n))
    srcr = jnp.concatenate([src, padrow])
    dstr = jnp.concatenate([dst, padrow])

    o4, dn = _make_sc_kernel(NT, NBLK)(y, asd, srcr, dstr)

    denb = jnp.broadcast_to(dn[:, :, None], (H, NT, CW))
    bgb = jnp.stack([bias, bn_gamma, bn_beta]).reshape(3, NCH, 1, 2 * CW)
    bgb = jnp.broadcast_to(bgb, (3, NCH, 8, 2 * CW))

    out = pl.pallas_call(
        _make_bn_body(NT, n),
        grid=(NCH, 2),
        in_specs=[pl.BlockSpec((2, NT, CW), lambda c4, p: (c4, 0, 0)),
                  pl.BlockSpec((1, NT, CW), lambda c4, p: (c4 // 2, 0, 0)),
                  pl.BlockSpec((3, 1, 8, 2 * CW),
                               lambda c4, p: (0, c4, 0, 0))],
        out_specs=pl.BlockSpec((n, 2 * CW), lambda c4, p: (0, c4)),
        out_shape=jax.ShapeDtypeStruct((n, H * OUT), jnp.float32),
        scratch_shapes=[pltpu.VMEM((8, 2 * CW), jnp.float32)],
    )(o4, denb, bgb)
    return out


# speedup vs baseline: 39.1533x; 1.2501x over previous
import jax
import jax.numpy as jnp
from jax import lax
from jax.experimental import pallas as pl
from jax.experimental.pallas import tpu as pltpu
from jax.experimental.pallas import tpu_sc as plsc

NEG = 0.2
_INTERP = False
B = 16
NB = 6
NTILES = 16
CW = 64
NCH = 4


def _mm_body(x_ref, w_ref, o_ref):
    y = jnp.dot(x_ref[...], w_ref[...], preferred_element_type=jnp.float32)
    o_ref[0] = y[:, :CW]
    o_ref[1] = y[:, CW:]


def _bcast_lane(v, i):
    dnums = lax.GatherDimensionNumbers(
        offset_dims=(), collapsed_slice_dims=(0,), start_index_map=(0,))
    idx = jnp.full((16, 1), i, jnp.int32)
    return lax.gather(v, idx, dnums, (1,),
                      mode=lax.GatherScatterMode.PROMISE_IN_BOUNDS)


def _make_sc_kernel(NT, NBLK):
    RPT = NT // NTILES
    EPT = NBLK * B

    def _sc_body(h4, asd, srcr, dstr, o4, dn,
                 src_v, dst_v, ex_v, as_t, ad_t, rows, sbuf, zbuf, zrow,
                 out_acc, den_acc, sem_g, sem_s, sem_d):
        c = lax.axis_index("c")
        s = lax.axis_index("s")

        pltpu.sync_copy(srcr.at[pl.ds(s * EPT, EPT)], src_v)
        pltpu.sync_copy(dstr.at[pl.ds(s * EPT, EPT)], dst_v)
        pltpu.sync_copy(asd.at[pl.ds(c * NT, NT)], as_t)
        pltpu.sync_copy(asd.at[pl.ds((c + 2) * NT, NT)], ad_t)

        zero16 = jnp.zeros((16,), jnp.float32)
        for i in range(64):
            for q in range(CW // 16):
                zbuf[i, pl.ds(q * 16, 16)] = zero16
        for i in range(RPT // 16):
            zrow[pl.ds(i * 16, 16)] = zero16
        pltpu.sync_copy(zrow, den_acc.at[pl.ds(s * RPT, RPT)])

        NGRP = NBLK // NB
        for jj in range(NCH):
            plane = c * NCH + jj
            for k in range(RPT // 64):
                pltpu.sync_copy(zbuf, out_acc.at[pl.ds(s * RPT + k * 64, 64)])
            plsc.subcore_barrier()

            for k in range(NB):
                pltpu.async_copy(h4.at[plane].at[src_v[pl.ds(k * B, B)]],
                                 rows.at[k], sem_g.at[k])

            def group_body(g, carry):
                for k in range(NB):
                    b = g * NB + k
                    di = dst_v[pl.ds(b * B, B)]
                    pltpu.make_async_copy(h4.at[plane].at[di], rows.at[k],
                                          sem_g.at[k]).wait()
                    if jj == 0:
                        si = src_v[pl.ds(b * B, B)]
                        asv = plsc.load_gather(as_t, [si])
                        adv = plsc.load_gather(ad_t, [di])
                        e = asv + adv
                        e = jnp.where(e > 0, e, NEG * e)
                        ex = jnp.exp(e)
                        ex_v[pl.ds(b * B, B)] = ex
                        pltpu.async_copy(ex_v.at[pl.ds(b * B, B)],
                                         den_acc.at[di], sem_d, add=True)
                    else:
                        ex = ex_v[pl.ds(b * B, B)]

                    @pl.when(g > 0)
                    def _():
                        pltpu.make_async_copy(sbuf.at[k], out_acc.at[di],
                                              sem_s.at[k]).wait()

                    for i in range(B):
                        bi = _bcast_lane(ex, i)
                        for q in range(CW // 16):
                            sl = pl.ds(q * 16, 16)
                            sbuf[k, i, sl] = rows[k, i, sl] * bi

                    @pl.when(g < NGRP - 1)
                    def _():
                        si2 = src_v[pl.ds((b + NB) * B, B)]
                        pltpu.async_copy(h4.at[plane].at[si2], rows.at[k],
                                         sem_g.at[k])

                    pltpu.async_copy(sbuf.at[k], out_acc.at[di], sem_s.at[k],
                                     add=True)
                return carry

            lax.fori_loop(0, NGRP, group_body, 0)
            if jj == 0:
                pltpu.make_async_copy(asd.at[pl.ds(0, EPT)], ex_v,
                                      sem_d).wait()
            for k in range(NB):
                pltpu.make_async_copy(sbuf.at[k],
                                      out_acc.at[dst_v[pl.ds(k * B, B)]],
                                      sem_s.at[k]).wait()
            plsc.subcore_barrier()
            pltpu.sync_copy(out_acc.at[pl.ds(s * RPT, RPT)],
                            o4.at[plane].at[pl.ds(s * RPT, RPT)])
            if jj == 0:
                pltpu.sync_copy(den_acc.at[pl.ds(s * RPT, RPT)],
                                dn.at[c].at[pl.ds(s * RPT, RPT)])
            plsc.subcore_barrier()

    mesh = plsc.VectorSubcoreMesh(core_axis_name="c", subcore_axis_name="s",
                                  num_cores=2, num_subcores=NTILES)
    return pl.kernel(
        _sc_body,
        out_type=[jax.ShapeDtypeStruct((2 * NCH, NT, CW), jnp.float32),
                  jax.ShapeDtypeStruct((2, NT), jnp.float32)],
        mesh=mesh,
        interpret=_INTERP,
        compiler_params=pltpu.CompilerParams(
            needs_layout_passes=False, use_tc_tiling_on_sc=False),
        scratch_types=[
            pltpu.VMEM((EPT,), jnp.int32),
            pltpu.VMEM((EPT,), jnp.int32),
            pltpu.VMEM((EPT,), jnp.float32),
            pltpu.VMEM((NT,), jnp.float32),
            pltpu.VMEM((NT,), jnp.float32),
            pltpu.VMEM((NB, B, CW), jnp.float32),
            pltpu.VMEM((NB, B, CW), jnp.float32),
            pltpu.VMEM((64, CW), jnp.float32),
            pltpu.VMEM((NT // NTILES,), jnp.float32),
            pltpu.VMEM_SHARED((NT, CW), jnp.float32),
            pltpu.VMEM_SHARED((NT,), jnp.float32),
            pltpu.SemaphoreType.DMA((NB,)),
            pltpu.SemaphoreType.DMA((NB,)),
            pltpu.SemaphoreType.DMA,
        ],
    )


def _make_bn_body(NT, n):
    def _bn_body(o4_ref, den_ref, bgb_ref, out_ref, st_ref):
        p = pl.program_id(1)
        den = den_ref[0]
        bias = bgb_ref[0, 0, 0:1, :]
        deni = 1.0 / (den + 1e-16)
        tmp = jnp.concatenate([o4_ref[0] * deni, o4_ref[1] * deni],
                              axis=1) + bias
        rowid = lax.broadcasted_iota(jnp.int32, (NT, 1), 0)
        tmpm = jnp.where(rowid < n, tmp, 0.0)

        @pl.when(p == 0)
        def _():
            st_ref[0:1, :] = jnp.sum(tmpm, axis=0, keepdims=True)
            st_ref[1:2, :] = jnp.sum(tmpm * tmpm, axis=0, keepdims=True)

        @pl.when(p == 1)
        def _():
            mean = st_ref[0:1, :] / n
            var = st_ref[1:2, :] / n - mean * mean
            gamma = bgb_ref[1, 0, 0:1, :]
            beta = bgb_ref[2, 0, 0:1, :]
            y = (tmp - mean) * lax.rsqrt(var + 1e-5) * gamma + beta
            out_ref[...] = jnp.maximum(y[:n, :], 0.0)

    return _bn_body


@jax.jit
def kernel(x, edge_index, W, att_src, att_dst, bias, bn_gamma, bn_beta):
    n, IN = x.shape
    H = att_src.shape[0]
    OUT = att_src.shape[1]
    e = edge_index.shape[1]
    NT = ((n + 255) // 256) * 256
    EE = e + n
    NBLK = -(-EE // (NTILES * B))
    NBLK = ((NBLK + NB - 1) // NB) * NB
    EP = NTILES * NBLK * B

    W3 = W.reshape(IN, H, OUT)
    Ws = (W3 * att_src[None]).sum(-1)
    Wd = (W3 * att_dst[None]).sum(-1)
    extra = jnp.zeros((IN, 2 * CW), jnp.float32)
    extra = extra.at[:, 0:H].set(Ws).at[:, H:2 * H].set(Wd)
    w_aug = jnp.concatenate([W, extra], axis=1)
    x_pad = jnp.pad(x, ((0, NT - n), (0, 0)))

    NCB = w_aug.shape[1] // CW
    y = pl.pallas_call(
        _mm_body,
        grid=(NT // 1024, NCB // 2),
        in_specs=[pl.BlockSpec((1024, IN), lambda i, j: (i, 0)),
                  pl.BlockSpec((IN, 2 * CW), lambda i, j: (0, j))],
        out_specs=pl.BlockSpec((2, 1024, CW), lambda i, j: (j, i, 0)),
        out_shape=jax.ShapeDtypeStruct((NCB, NT, CW), jnp.float32),
    )(x_pad, w_aug)

    asd = y[2 * NCH, :, 0:2 * H].T.reshape(-1)

    loops = jnp.arange(n, dtype=jnp.int32)
    src = jnp.concatenate([edge_index[0], loops])
    dst = jnp.concatenate([edge_index[1], loops])
    padrow = n + (jnp.arange(EP - EE, dtype=jnp.int32) % (NT - n))
    srcr = jnp.concatenate([src, padrow])
    dstr = jnp.concatenate([dst, padrow])

    o4, dn = _make_sc_kernel(NT, NBLK)(y, asd, srcr, dstr)

    denb = jnp.broadcast_to(dn[:, :, None], (H, NT, CW))
    bgb = jnp.stack([bias, bn_gamma, bn_beta]).reshape(3, NCH, 1, 2 * CW)
    bgb = jnp.broadcast_to(bgb, (3, NCH, 8, 2 * CW))

    out = pl.pallas_call(
        _make_bn_body(NT, n),
        grid=(NCH, 2),
        in_specs=[pl.BlockSpec((2, NT, CW), lambda c4, p: (c4, 0, 0)),
                  pl.BlockSpec((1, NT, CW), lambda c4, p: (c4 // 2, 0, 0)),
                  pl.BlockSpec((3, 1, 8, 2 * CW),
                               lambda c4, p: (0, c4, 0, 0))],
        out_specs=pl.BlockSpec((n, 2 * CW), lambda c4, p: (0, c4)),
        out_shape=jax.ShapeDtypeStruct((n, H * OUT), jnp.float32),
        scratch_shapes=[pltpu.VMEM((8, 2 * CW), jnp.float32)],
    )(o4, denb, bgb)
    return out


# speedup vs baseline: 41.4838x; 1.0595x over previous
import jax
import jax.numpy as jnp
from jax import lax
from jax.experimental import pallas as pl
from jax.experimental.pallas import tpu as pltpu
from jax.experimental.pallas import tpu_sc as plsc

NEG = 0.2
_INTERP = False
B = 32
NB = 4
NTILES = 16
CW = 64
NCH = 4


def _mm_body(x_ref, w_ref, o_ref):
    y = jnp.dot(x_ref[...], w_ref[...], preferred_element_type=jnp.float32)
    o_ref[0] = y[:, :CW]
    o_ref[1] = y[:, CW:]


def _bcast_lane(v, i):
    dnums = lax.GatherDimensionNumbers(
        offset_dims=(), collapsed_slice_dims=(0,), start_index_map=(0,))
    idx = jnp.full((16, 1), i, jnp.int32)
    return lax.gather(v, idx, dnums, (1,),
                      mode=lax.GatherScatterMode.PROMISE_IN_BOUNDS)


def _make_sc_kernel(NT, NBLK):
    RPT = NT // NTILES
    EPT = NBLK * B

    def _sc_body(h4, asd, srcr, dstr, o4, dn,
                 src_v, dst_v, ex_v, as_t, ad_t, rows, sbuf, zbuf, zrow,
                 out_acc, den_acc, sem_g, sem_s, sem_d):
        c = lax.axis_index("c")
        s = lax.axis_index("s")

        pltpu.sync_copy(srcr.at[pl.ds(s * EPT, EPT)], src_v)
        pltpu.sync_copy(dstr.at[pl.ds(s * EPT, EPT)], dst_v)
        pltpu.sync_copy(asd.at[pl.ds(c * NT, NT)], as_t)
        pltpu.sync_copy(asd.at[pl.ds((c + 2) * NT, NT)], ad_t)

        zero16 = jnp.zeros((16,), jnp.float32)
        for i in range(64):
            for q in range(CW // 16):
                zbuf[i, pl.ds(q * 16, 16)] = zero16
        for i in range(RPT // 16):
            zrow[pl.ds(i * 16, 16)] = zero16
        pltpu.sync_copy(zrow, den_acc.at[pl.ds(s * RPT, RPT)])

        NGRP = NBLK // NB
        for jj in range(NCH):
            plane = c * NCH + jj
            for k in range(RPT // 64):
                pltpu.sync_copy(zbuf, out_acc.at[pl.ds(s * RPT + k * 64, 64)])
            plsc.subcore_barrier()

            for k in range(NB):
                pltpu.async_copy(
                    h4.at[plane].at[src_v.at[pl.ds(k * B, B)]],
                    rows.at[k], sem_g.at[k])

            def group_body(g, carry):
                for k in range(NB):
                    b = g * NB + k
                    pltpu.make_async_copy(
                        h4.at[plane].at[src_v.at[pl.ds(b * B, B)]],
                        rows.at[k], sem_g.at[k]).wait()
                    dis = []
                    exs = []
                    for u in range(B // 16):
                        di = dst_v[pl.ds(b * B + u * 16, 16)]
                        dis.append(di)
                        if jj == 0:
                            si = src_v[pl.ds(b * B + u * 16, 16)]
                            asv = plsc.load_gather(as_t, [si])
                            adv = plsc.load_gather(ad_t, [di])
                            e = asv + adv
                            e = jnp.where(e > 0, e, NEG * e)
                            ex = jnp.exp(e)
                            ex_v[pl.ds(b * B + u * 16, 16)] = ex
                            pltpu.async_copy(
                                ex_v.at[pl.ds(b * B + u * 16, 16)],
                                den_acc.at[di], sem_d, add=True)
                        else:
                            ex = ex_v[pl.ds(b * B + u * 16, 16)]
                        exs.append(ex)

                    @pl.when(g > 0)
                    def _():
                        for u in range(B // 16):
                            pltpu.make_async_copy(
                                sbuf.at[k].at[pl.ds(u * 16, 16)],
                                out_acc.at[dis[u]], sem_s.at[k]).wait()

                    for u in range(B // 16):
                        for i in range(16):
                            bi = _bcast_lane(exs[u], i)
                            for q in range(CW // 16):
                                sl = pl.ds(q * 16, 16)
                                sbuf[k, u * 16 + i, sl] = \
                                    rows[k, u * 16 + i, sl] * bi

                    @pl.when(g < NGRP - 1)
                    def _():
                        pltpu.async_copy(
                            h4.at[plane].at[
                                src_v.at[pl.ds((b + NB) * B, B)]],
                            rows.at[k], sem_g.at[k])

                    for u in range(B // 16):
                        pltpu.async_copy(sbuf.at[k].at[pl.ds(u * 16, 16)],
                                         out_acc.at[dis[u]], sem_s.at[k],
                                         add=True)
                return carry

            lax.fori_loop(0, NGRP, group_body, 0)
            if jj == 0:
                pltpu.make_async_copy(asd.at[pl.ds(0, EPT)], ex_v,
                                      sem_d).wait()
            for k in range(NB):
                for u in range(B // 16):
                    pltpu.make_async_copy(
                        sbuf.at[k].at[pl.ds(u * 16, 16)],
                        out_acc.at[dst_v[pl.ds(u * 16, 16)]],
                        sem_s.at[k]).wait()
            plsc.subcore_barrier()
            pltpu.sync_copy(out_acc.at[pl.ds(s * RPT, RPT)],
                            o4.at[plane].at[pl.ds(s * RPT, RPT)])
            if jj == 0:
                pltpu.sync_copy(den_acc.at[pl.ds(s * RPT, RPT)],
                                dn.at[c].at[pl.ds(s * RPT, RPT)])
            plsc.subcore_barrier()

    mesh = plsc.VectorSubcoreMesh(core_axis_name="c", subcore_axis_name="s",
                                  num_cores=2, num_subcores=NTILES)
    return pl.kernel(
        _sc_body,
        out_type=[jax.ShapeDtypeStruct((2 * NCH, NT, CW), jnp.float32),
                  jax.ShapeDtypeStruct((2, NT), jnp.float32)],
        mesh=mesh,
        interpret=_INTERP,
        compiler_params=pltpu.CompilerParams(
            needs_layout_passes=False, use_tc_tiling_on_sc=False),
        scratch_types=[
            pltpu.VMEM((EPT,), jnp.int32),
            pltpu.VMEM((EPT,), jnp.int32),
            pltpu.VMEM((EPT,), jnp.float32),
            pltpu.VMEM((NT,), jnp.float32),
            pltpu.VMEM((NT,), jnp.float32),
            pltpu.VMEM((NB, B, CW), jnp.float32),
            pltpu.VMEM((NB, B, CW), jnp.float32),
            pltpu.VMEM((64, CW), jnp.float32),
            pltpu.VMEM((NT // NTILES,), jnp.float32),
            pltpu.VMEM_SHARED((NT, CW), jnp.float32),
            pltpu.VMEM_SHARED((NT,), jnp.float32),
            pltpu.SemaphoreType.DMA((NB,)),
            pltpu.SemaphoreType.DMA((NB,)),
            pltpu.SemaphoreType.DMA,
        ],
    )


def _make_bn_body(NT, n):
    def _bn_body(o4_ref, den_ref, bgb_ref, out_ref, st_ref):
        p = pl.program_id(1)
        den = den_ref[0]
        bias = bgb_ref[0, 0, 0:1, :]
        deni = 1.0 / (den + 1e-16)
        tmp = jnp.concatenate([o4_ref[0] * deni, o4_ref[1] * deni],
                              axis=1) + bias
        rowid = lax.broadcasted_iota(jnp.int32, (NT, 1), 0)
        tmpm = jnp.where(rowid < n, tmp, 0.0)

        @pl.when(p == 0)
        def _():
            st_ref[0:1, :] = jnp.sum(tmpm, axis=0, keepdims=True)
            st_ref[1:2, :] = jnp.sum(tmpm * tmpm, axis=0, keepdims=True)

        @pl.when(p == 1)
        def _():
            mean = st_ref[0:1, :] / n
            var = st_ref[1:2, :] / n - mean * mean
            gamma = bgb_ref[1, 0, 0:1, :]
            beta = bgb_ref[2, 0, 0:1, :]
            y = (tmp - mean) * lax.rsqrt(var + 1e-5) * gamma + beta
            out_ref[...] = jnp.maximum(y[:n, :], 0.0)

    return _bn_body


@jax.jit
def kernel(x, edge_index, W, att_src, att_dst, bias, bn_gamma, bn_beta):
    n, IN = x.shape
    H = att_src.shape[0]
    OUT = att_src.shape[1]
    e = edge_index.shape[1]
    NT = ((n + 255) // 256) * 256
    EE = e + n
    NBLK = -(-EE // (NTILES * B))
    NBLK = ((NBLK + NB - 1) // NB) * NB
    EP = NTILES * NBLK * B

    W3 = W.reshape(IN, H, OUT)
    Ws = (W3 * att_src[None]).sum(-1)
    Wd = (W3 * att_dst[None]).sum(-1)
    extra = jnp.zeros((IN, 2 * CW), jnp.float32)
    extra = extra.at[:, 0:H].set(Ws).at[:, H:2 * H].set(Wd)
    w_aug = jnp.concatenate([W, extra], axis=1)
    x_pad = jnp.pad(x, ((0, NT - n), (0, 0)))

    NCB = w_aug.shape[1] // CW
    y = pl.pallas_call(
        _mm_body,
        grid=(NT // 1024, NCB // 2),
        in_specs=[pl.BlockSpec((1024, IN), lambda i, j: (i, 0)),
                  pl.BlockSpec((IN, 2 * CW), lambda i, j: (0, j))],
        out_specs=pl.BlockSpec((2, 1024, CW), lambda i, j: (j, i, 0)),
        out_shape=jax.ShapeDtypeStruct((NCB, NT, CW), jnp.float32),
    )(x_pad, w_aug)

    asd = y[2 * NCH, :, 0:2 * H].T.reshape(-1)

    loops = jnp.arange(n, dtype=jnp.int32)
    src = jnp.concatenate([edge_index[0], loops])
    dst = jnp.concatenate([edge_index[1], loops])
    padrow = n + (jnp.arange(EP - EE, dtype=jnp.int32) % (NT - n))
    srcr = jnp.concatenate([src, padrow])
    dstr = jnp.concatenate([dst, padrow])

    o4, dn = _make_sc_kernel(NT, NBLK)(y, asd, srcr, dstr)

    denb = jnp.broadcast_to(dn[:, :, None], (H, NT, CW))
    bgb = jnp.stack([bias, bn_gamma, bn_beta]).reshape(3, NCH, 1, 2 * CW)
    bgb = jnp.broadcast_to(bgb, (3, NCH, 8, 2 * CW))

    out = pl.pallas_call(
        _make_bn_body(NT, n),
        grid=(NCH, 2),
        in_specs=[pl.BlockSpec((2, NT, CW), lambda c4, p: (c4, 0, 0)),
                  pl.BlockSpec((1, NT, CW), lambda c4, p: (c4 // 2, 0, 0)),
                  pl.BlockSpec((3, 1, 8, 2 * CW),
                               lambda c4, p: (0, c4, 0, 0))],
        out_specs=pl.BlockSpec((n, 2 * CW), lambda c4, p: (0, c4)),
        out_shape=jax.ShapeDtypeStruct((n, H * OUT), jnp.float32),
        scratch_shapes=[pltpu.VMEM((8, 2 * CW), jnp.float32)],
    )(o4, denb, bgb)
    return out


# speedup vs baseline: 47.0798x; 1.1349x over previous
import jax
import jax.numpy as jnp
from jax import lax
from jax.experimental import pallas as pl
from jax.experimental.pallas import tpu as pltpu
from jax.experimental.pallas import tpu_sc as plsc

NEG = 0.2
_INTERP = False
B = 32
NB = 6
NTILES = 16
CW = 64
NCH = 4


def _mm_body(x_ref, w_ref, o_ref):
    y = jnp.dot(x_ref[...], w_ref[...], preferred_element_type=jnp.float32)
    o_ref[0] = y[:, :CW]
    o_ref[1] = y[:, CW:]


def _bcast_lane(v, i):
    dnums = lax.GatherDimensionNumbers(
        offset_dims=(), collapsed_slice_dims=(0,), start_index_map=(0,))
    idx = jnp.full((16, 1), i, jnp.int32)
    return lax.gather(v, idx, dnums, (1,),
                      mode=lax.GatherScatterMode.PROMISE_IN_BOUNDS)


def _make_sc_kernel(NT, NBLK):
    RPT = NT // NTILES
    EPT = NBLK * B

    def _sc_body(h4, asd, srcr, dstr, o4, dn,
                 src_v, dst_v, ex_v, as_t, ad_t, rows, sbuf, zbuf, zrow,
                 out_acc, den_acc, sem_g, sem_s, sem_d):
        c = lax.axis_index("c")
        s = lax.axis_index("s")

        pltpu.sync_copy(srcr.at[pl.ds(s * EPT, EPT)], src_v)
        pltpu.sync_copy(dstr.at[pl.ds(s * EPT, EPT)], dst_v)
        pltpu.sync_copy(asd.at[pl.ds(c * NT, NT)], as_t)
        pltpu.sync_copy(asd.at[pl.ds((c + 2) * NT, NT)], ad_t)

        zero16 = jnp.zeros((16,), jnp.float32)
        for i in range(64):
            for q in range(CW // 16):
                zbuf[i, pl.ds(q * 16, 16)] = zero16
        for i in range(RPT // 16):
            zrow[pl.ds(i * 16, 16)] = zero16
        pltpu.sync_copy(zrow, den_acc.at[pl.ds(s * RPT, RPT)])

        NGRP = NBLK // NB
        for jj in range(NCH):
            plane = c * NCH + jj
            for k in range(RPT // 64):
                pltpu.sync_copy(zbuf, out_acc.at[pl.ds(s * RPT + k * 64, 64)])
            plsc.subcore_barrier()

            for k in range(NB):
                pltpu.async_copy(
                    h4.at[plane].at[src_v.at[pl.ds(k * B, B)]],
                    rows.at[k], sem_g.at[k])

            def group_body(g, carry):
                for k in range(NB):
                    b = g * NB + k
                    pltpu.make_async_copy(
                        h4.at[plane].at[src_v.at[pl.ds(b * B, B)]],
                        rows.at[k], sem_g.at[k]).wait()
                    dis = []
                    exs = []
                    for u in range(B // 16):
                        di = dst_v[pl.ds(b * B + u * 16, 16)]
                        dis.append(di)
                        if jj == 0:
                            si = src_v[pl.ds(b * B + u * 16, 16)]
                            asv = plsc.load_gather(as_t, [si])
                            adv = plsc.load_gather(ad_t, [di])
                            e = asv + adv
                            e = jnp.where(e > 0, e, NEG * e)
                            ex = jnp.exp(e)
                            ex_v[pl.ds(b * B + u * 16, 16)] = ex
                            pltpu.async_copy(
                                ex_v.at[pl.ds(b * B + u * 16, 16)],
                                den_acc.at[di], sem_d, add=True)
                        else:
                            ex = ex_v[pl.ds(b * B + u * 16, 16)]
                        exs.append(ex)

                    @pl.when(g > 0)
                    def _():
                        for u in range(B // 16):
                            pltpu.make_async_copy(
                                sbuf.at[k].at[pl.ds(u * 16, 16)],
                                out_acc.at[dis[u]], sem_s.at[k]).wait()

                    for u in range(B // 16):
                        for i in range(16):
                            bi = _bcast_lane(exs[u], i)
                            for q in range(CW // 16):
                                sl = pl.ds(q * 16, 16)
                                sbuf[k, u * 16 + i, sl] = \
                                    rows[k, u * 16 + i, sl] * bi

                    @pl.when(g < NGRP - 1)
                    def _():
                        pltpu.async_copy(
                            h4.at[plane].at[
                                src_v.at[pl.ds((b + NB) * B, B)]],
                            rows.at[k], sem_g.at[k])

                    for u in range(B // 16):
                        pltpu.async_copy(sbuf.at[k].at[pl.ds(u * 16, 16)],
                                         out_acc.at[dis[u]], sem_s.at[k],
                                         add=True)
                return carry

            lax.fori_loop(0, NGRP, group_body, 0)
            if jj == 0:
                pltpu.make_async_copy(asd.at[pl.ds(0, EPT)], ex_v,
                                      sem_d).wait()
            for k in range(NB):
                for u in range(B // 16):
                    pltpu.make_async_copy(
                        sbuf.at[k].at[pl.ds(u * 16, 16)],
                        out_acc.at[dst_v[pl.ds(u * 16, 16)]],
                        sem_s.at[k]).wait()
            plsc.subcore_barrier()
            pltpu.sync_copy(out_acc.at[pl.ds(s * RPT, RPT)],
                            o4.at[plane].at[pl.ds(s * RPT, RPT)])
            if jj == 0:
                pltpu.sync_copy(den_acc.at[pl.ds(s * RPT, RPT)],
                                dn.at[c].at[pl.ds(s * RPT, RPT)])
            plsc.subcore_barrier()

    mesh = plsc.VectorSubcoreMesh(core_axis_name="c", subcore_axis_name="s",
                                  num_cores=2, num_subcores=NTILES)
    return pl.kernel(
        _sc_body,
        out_type=[jax.ShapeDtypeStruct((2 * NCH, NT, CW), jnp.float32),
                  jax.ShapeDtypeStruct((2, NT), jnp.float32)],
        mesh=mesh,
        interpret=_INTERP,
        compiler_params=pltpu.CompilerParams(
            needs_layout_passes=False, use_tc_tiling_on_sc=False),
        scratch_types=[
            pltpu.VMEM((EPT,), jnp.int32),
            pltpu.VMEM((EPT,), jnp.int32),
            pltpu.VMEM((EPT,), jnp.float32),
            pltpu.VMEM((NT,), jnp.float32),
            pltpu.VMEM((NT,), jnp.float32),
            pltpu.VMEM((NB, B, CW), jnp.float32),
            pltpu.VMEM((NB, B, CW), jnp.float32),
            pltpu.VMEM((64, CW), jnp.float32),
            pltpu.VMEM((NT // NTILES,), jnp.float32),
            pltpu.VMEM_SHARED((NT, CW), jnp.float32),
            pltpu.VMEM_SHARED((NT,), jnp.float32),
            pltpu.SemaphoreType.DMA((NB,)),
            pltpu.SemaphoreType.DMA((NB,)),
            pltpu.SemaphoreType.DMA,
        ],
    )


def _make_bn_body(NT, n):
    def _bn_body(o4_ref, den_ref, bgb_ref, out_ref, st_ref):
        p = pl.program_id(1)
        den = den_ref[0]
        bias = bgb_ref[0, 0, 0:1, :]
        deni = 1.0 / (den + 1e-16)
        tmp = jnp.concatenate([o4_ref[0] * deni, o4_ref[1] * deni],
                              axis=1) + bias
        rowid = lax.broadcasted_iota(jnp.int32, (NT, 1), 0)
        tmpm = jnp.where(rowid < n, tmp, 0.0)

        @pl.when(p == 0)
        def _():
            st_ref[0:1, :] = jnp.sum(tmpm, axis=0, keepdims=True)
            st_ref[1:2, :] = jnp.sum(tmpm * tmpm, axis=0, keepdims=True)

        @pl.when(p == 1)
        def _():
            mean = st_ref[0:1, :] / n
            var = st_ref[1:2, :] / n - mean * mean
            gamma = bgb_ref[1, 0, 0:1, :]
            beta = bgb_ref[2, 0, 0:1, :]
            y = (tmp - mean) * lax.rsqrt(var + 1e-5) * gamma + beta
            out_ref[...] = jnp.maximum(y[:n, :], 0.0)

    return _bn_body


@jax.jit
def kernel(x, edge_index, W, att_src, att_dst, bias, bn_gamma, bn_beta):
    n, IN = x.shape
    H = att_src.shape[0]
    OUT = att_src.shape[1]
    e = edge_index.shape[1]
    NT = ((n + 255) // 256) * 256
    EE = e + n
    NBLK = -(-EE // (NTILES * B))
    NBLK = ((NBLK + NB - 1) // NB) * NB
    EP = NTILES * NBLK * B

    W3 = W.reshape(IN, H, OUT)
    Ws = (W3 * att_src[None]).sum(-1)
    Wd = (W3 * att_dst[None]).sum(-1)
    extra = jnp.zeros((IN, 2 * CW), jnp.float32)
    extra = extra.at[:, 0:H].set(Ws).at[:, H:2 * H].set(Wd)
    w_aug = jnp.concatenate([W, extra], axis=1).astype(jnp.bfloat16)
    x_pad = jnp.pad(x, ((0, NT - n), (0, 0))).astype(jnp.bfloat16)

    NCB = w_aug.shape[1] // CW
    y = pl.pallas_call(
        _mm_body,
        grid=(NT // 1024, NCB // 2),
        in_specs=[pl.BlockSpec((1024, IN), lambda i, j: (i, 0)),
                  pl.BlockSpec((IN, 2 * CW), lambda i, j: (0, j))],
        out_specs=pl.BlockSpec((2, 1024, CW), lambda i, j: (j, i, 0)),
        out_shape=jax.ShapeDtypeStruct((NCB, NT, CW), jnp.float32),
    )(x_pad, w_aug)

    asd = y[2 * NCH, :, 0:2 * H].T.reshape(-1)

    loops = jnp.arange(n, dtype=jnp.int32)
    src = jnp.concatenate([edge_index[0], loops])
    dst = jnp.concatenate([edge_index[1], loops])
    padrow = n + (jnp.arange(EP - EE, dtype=jnp.int32) % (NT - n))
    srcr = jnp.concatenate([src, padrow])
    dstr = jnp.concatenate([dst, padrow])

    o4, dn = _make_sc_kernel(NT, NBLK)(y, asd, srcr, dstr)

    denb = jnp.broadcast_to(dn[:, :, None], (H, NT, CW))
    bgb = jnp.stack([bias, bn_gamma, bn_beta]).reshape(3, NCH, 1, 2 * CW)
    bgb = jnp.broadcast_to(bgb, (3, NCH, 8, 2 * CW))

    out = pl.pallas_call(
        _make_bn_body(NT, n),
        grid=(NCH, 2),
        in_specs=[pl.BlockSpec((2, NT, CW), lambda c4, p: (c4, 0, 0)),
                  pl.BlockSpec((1, NT, CW), lambda c4, p: (c4 // 2, 0, 0)),
                  pl.BlockSpec((3, 1, 8, 2 * CW),
                               lambda c4, p: (0, c4, 0, 0))],
        out_specs=pl.BlockSpec((n, 2 * CW), lambda c4, p: (0, c4)),
        out_shape=jax.ShapeDtypeStruct((n, H * OUT), jnp.float32),
        scratch_shapes=[pltpu.VMEM((8, 2 * CW), jnp.float32)],
    )(o4, denb, bgb)
    return out


# speedup vs baseline: 55.6802x; 1.1827x over previous
import jax
import jax.numpy as jnp
from jax import lax
from jax.experimental import pallas as pl
from jax.experimental.pallas import tpu as pltpu
from jax.experimental.pallas import tpu_sc as plsc

NEG = 0.2
_INTERP = False
B = 32
NB = 6
NTILES = 16
CW = 64
NCH = 4


def _mm_body(x_ref, w_ref, o_ref):
    y = jnp.dot(x_ref[...], w_ref[...], preferred_element_type=jnp.float32)
    o_ref[0, :, 0:CW] = y[:, :CW]
    o_ref[1, :, 0:CW] = y[:, CW:]


def _bcast_lane(v, i):
    dnums = lax.GatherDimensionNumbers(
        offset_dims=(), collapsed_slice_dims=(0,), start_index_map=(0,))
    idx = jnp.full((16, 1), i, jnp.int32)
    return lax.gather(v, idx, dnums, (1,),
                      mode=lax.GatherScatterMode.PROMISE_IN_BOUNDS)


def _make_sc_kernel(NT, NBLK):
    RPT = NT // NTILES
    EPT = NBLK * B

    def _sc_body(h4, asd, src2r, dstr, o4, dn,
                 src2_v, dst_v, ex_v, as_t, ad_t, rows, sbuf, zbuf,
                 zrow, out_acc, den_acc, sem_g, sem_s, sem_d):
        c = lax.axis_index("c")
        s = lax.axis_index("s")

        pltpu.sync_copy(src2r.at[pl.ds(s * EPT, EPT)], src2_v)
        pltpu.sync_copy(dstr.at[pl.ds(s * EPT, EPT)], dst_v)
        pltpu.sync_copy(asd.at[pl.ds(c * NT, NT)], as_t)
        pltpu.sync_copy(asd.at[pl.ds((c + 2) * NT, NT)], ad_t)

        zero16 = jnp.zeros((16,), jnp.float32)
        for i in range(64):
            for q in range(CW // 16):
                zbuf[i, pl.ds(q * 16, 16)] = zero16
        for i in range(RPT // 16):
            zrow[pl.ds(i * 16, 16)] = zero16
        pltpu.sync_copy(zrow, den_acc.at[pl.ds(s * RPT, RPT)])

        NGRP = NBLK // NB
        for jj in range(NCH):
            plane = c * NCH + jj
            for k in range(RPT // 64):
                pltpu.sync_copy(zbuf, out_acc.at[pl.ds(s * RPT + k * 64, 64)])
            plsc.subcore_barrier()

            for k in range(NB):
                pltpu.async_copy(
                    h4.at[plane].at[src2_v.at[pl.ds(k * B, B)]],
                    rows.at[k], sem_g.at[k])

            def group_body(g, carry):
                for k in range(NB):
                    b = g * NB + k
                    pltpu.make_async_copy(
                        h4.at[plane].at[src2_v.at[pl.ds(b * B, B)]],
                        rows.at[k], sem_g.at[k]).wait()
                    dis = []
                    exs = []
                    for u in range(B // 16):
                        di = dst_v[pl.ds(b * B + u * 16, 16)]
                        dis.append(di)
                        if jj == 0:
                            si = lax.shift_right_logical(
                                src2_v[pl.ds(b * B + u * 16, 16)], 1)
                            asv = plsc.load_gather(as_t, [si])
                            adv = plsc.load_gather(ad_t, [di])
                            e = asv + adv
                            e = jnp.where(e > 0, e, NEG * e)
                            ex = jnp.exp(e)
                            ex_v[pl.ds(b * B + u * 16, 16)] = ex
                            pltpu.async_copy(
                                ex_v.at[pl.ds(b * B + u * 16, 16)],
                                den_acc.at[di], sem_d, add=True)
                        else:
                            ex = ex_v[pl.ds(b * B + u * 16, 16)]
                        exs.append(ex)

                    @pl.when(g > 0)
                    def _():
                        for u in range(B // 16):
                            pltpu.make_async_copy(
                                sbuf.at[k].at[pl.ds(u * 16, 16)],
                                out_acc.at[dis[u]], sem_s.at[k]).wait()

                    for u in range(B // 16):
                        for i in range(16):
                            bi = _bcast_lane(exs[u], i)
                            for q in range(CW // 16):
                                sl = pl.ds(q * 16, 16)
                                sbuf[k, u * 16 + i, sl] = \
                                    rows[k, u * 16 + i, sl] * bi

                    @pl.when(g < NGRP - 1)
                    def _():
                        pltpu.async_copy(
                            h4.at[plane].at[
                                src2_v.at[pl.ds((b + NB) * B, B)]],
                            rows.at[k], sem_g.at[k])

                    for u in range(B // 16):
                        pltpu.async_copy(sbuf.at[k].at[pl.ds(u * 16, 16)],
                                         out_acc.at[dis[u]], sem_s.at[k],
                                         add=True)
                return carry

            lax.fori_loop(0, NGRP, group_body, 0)
            if jj == 0:
                pltpu.make_async_copy(asd.at[pl.ds(0, EPT)], ex_v,
                                      sem_d).wait()
            for k in range(NB):
                for u in range(B // 16):
                    pltpu.make_async_copy(
                        sbuf.at[k].at[pl.ds(u * 16, 16)],
                        out_acc.at[dst_v[pl.ds(u * 16, 16)]],
                        sem_s.at[k]).wait()
            plsc.subcore_barrier()
            pltpu.sync_copy(out_acc.at[pl.ds(s * RPT, RPT)],
                            o4.at[plane].at[pl.ds(s * RPT, RPT),
                                            pl.ds(0, CW)])
            if jj == 0:
                pltpu.sync_copy(den_acc.at[pl.ds(s * RPT, RPT)],
                                dn.at[c].at[pl.ds(s * RPT, RPT)])
            plsc.subcore_barrier()

    mesh = plsc.VectorSubcoreMesh(core_axis_name="c", subcore_axis_name="s",
                                  num_cores=2, num_subcores=NTILES)
    return pl.kernel(
        _sc_body,
        out_type=[jax.ShapeDtypeStruct((2 * NCH, NT, 2 * CW), jnp.float32),
                  jax.ShapeDtypeStruct((2, NT), jnp.float32)],
        mesh=mesh,
        interpret=_INTERP,
        compiler_params=pltpu.CompilerParams(
            needs_layout_passes=False, use_tc_tiling_on_sc=False),
        scratch_types=[
            pltpu.VMEM((EPT,), jnp.int32),
            pltpu.VMEM((EPT,), jnp.int32),
            pltpu.VMEM((EPT,), jnp.float32),
            pltpu.VMEM((NT,), jnp.float32),
            pltpu.VMEM((NT,), jnp.float32),
            pltpu.VMEM((NB, B, CW), jnp.float32),
            pltpu.VMEM((NB, B, CW), jnp.float32),
            pltpu.VMEM((64, CW), jnp.float32),
            pltpu.VMEM((NT // NTILES,), jnp.float32),
            pltpu.VMEM_SHARED((NT, CW), jnp.float32),
            pltpu.VMEM_SHARED((NT,), jnp.float32),
            pltpu.SemaphoreType.DMA((NB,)),
            pltpu.SemaphoreType.DMA((NB,)),
            pltpu.SemaphoreType.DMA,
        ],
    )


def _make_bn_body(NT, n):
    def _bn_body(o4_ref, den_ref, bgb_ref, out_ref, st_ref):
        p = pl.program_id(1)
        den = den_ref[0]
        bias = bgb_ref[0, 0, 0:1, :]
        deni = 1.0 / (den + 1e-16)
        tmp = jnp.concatenate([o4_ref[0][:, :CW] * deni,
                               o4_ref[1][:, :CW] * deni], axis=1) + bias
        rowid = lax.broadcasted_iota(jnp.int32, (NT, 1), 0)
        tmpm = jnp.where(rowid < n, tmp, 0.0)

        @pl.when(p == 0)
        def _():
            st_ref[0:1, :] = jnp.sum(tmpm, axis=0, keepdims=True)
            st_ref[1:2, :] = jnp.sum(tmpm * tmpm, axis=0, keepdims=True)

        @pl.when(p == 1)
        def _():
            mean = st_ref[0:1, :] / n
            var = st_ref[1:2, :] / n - mean * mean
            gamma = bgb_ref[1, 0, 0:1, :]
            beta = bgb_ref[2, 0, 0:1, :]
            y = (tmp - mean) * lax.rsqrt(var + 1e-5) * gamma + beta
            out_ref[...] = jnp.maximum(y[:n, :], 0.0)

    return _bn_body


@jax.jit
def kernel(x, edge_index, W, att_src, att_dst, bias, bn_gamma, bn_beta):
    n, IN = x.shape
    H = att_src.shape[0]
    OUT = att_src.shape[1]
    e = edge_index.shape[1]
    NT = ((n + 255) // 256) * 256
    EE = e + n
    NBLK = -(-EE // (NTILES * B))
    NBLK = ((NBLK + NB - 1) // NB) * NB
    EP = NTILES * NBLK * B

    W3 = W.reshape(IN, H, OUT)
    Ws = (W3 * att_src[None]).sum(-1)
    Wd = (W3 * att_dst[None]).sum(-1)
    extra = jnp.zeros((IN, 2 * CW), jnp.float32)
    extra = extra.at[:, 0:H].set(Ws).at[:, H:2 * H].set(Wd)
    w_aug = jnp.concatenate([W, extra], axis=1).astype(jnp.bfloat16)
    x_pad = jnp.pad(x, ((0, NT - n), (0, 0))).astype(jnp.bfloat16)

    NCB = w_aug.shape[1] // CW
    y = pl.pallas_call(
        _mm_body,
        grid=(NT // 1024, NCB // 2),
        in_specs=[pl.BlockSpec((1024, IN), lambda i, j: (i, 0)),
                  pl.BlockSpec((IN, 2 * CW), lambda i, j: (0, j))],
        out_specs=pl.BlockSpec((2, 1024, 2 * CW), lambda i, j: (j, i, 0)),
        out_shape=jax.ShapeDtypeStruct((NCB, NT, 2 * CW), jnp.float32),
    )(x_pad, w_aug)

    asd = y[2 * NCH, :, 0:2 * H].T.reshape(-1)
    h4r = y.reshape(NCB, 2 * NT, CW)

    loops = jnp.arange(n, dtype=jnp.int32)
    src = jnp.concatenate([edge_index[0], loops])
    dst = jnp.concatenate([edge_index[1], loops])
    padrow = n + (jnp.arange(EP - EE, dtype=jnp.int32) % (NT - n))
    srcr = jnp.concatenate([src, padrow])
    dstr = jnp.concatenate([dst, padrow])
    src2r = srcr * 2

    o4, dn = _make_sc_kernel(NT, NBLK)(h4r, asd, src2r, dstr)

    denb = jnp.broadcast_to(dn[:, :, None], (H, NT, CW))
    bgb = jnp.stack([bias, bn_gamma, bn_beta]).reshape(3, NCH, 1, 2 * CW)
    bgb = jnp.broadcast_to(bgb, (3, NCH, 8, 2 * CW))

    out = pl.pallas_call(
        _make_bn_body(NT, n),
        grid=(NCH, 2),
        in_specs=[pl.BlockSpec((2, NT, 2 * CW), lambda c4, p: (c4, 0, 0)),
                  pl.BlockSpec((1, NT, CW), lambda c4, p: (c4 // 2, 0, 0)),
                  pl.BlockSpec((3, 1, 8, 2 * CW),
                               lambda c4, p: (0, c4, 0, 0))],
        out_specs=pl.BlockSpec((n, 2 * CW), lambda c4, p: (0, c4)),
        out_shape=jax.ShapeDtypeStruct((n, H * OUT), jnp.float32),
        scratch_shapes=[pltpu.VMEM((8, 2 * CW), jnp.float32)],
    )(o4, denb, bgb)
    return out
